# Initial kernel scaffold; baseline (speedup 1.0000x reference)
import functools

import jax
import jax.numpy as jnp
from jax.experimental import pallas as pl
from jax.experimental.pallas import tpu as pltpu
from jax.experimental.pallas import tpu_sc as plsc

_WINDOW = 128


@functools.cache
def _build(num_indices: int, dim: int, dtype):
    mesh = plsc.VectorSubcoreMesh(core_axis_name="core",
                                  subcore_axis_name="subcore")

    @functools.partial(
        pl.kernel,
        out_type=jax.ShapeDtypeStruct((num_indices, dim), dtype),
        mesh=mesh,
    )
    def gather_kernel(w_hbm, i_hbm, o_hbm):
        def body(i_vmem, o_vmem):
            pltpu.sync_copy(w_hbm.at[i_vmem.at[0]], o_vmem)

        pltpu.emit_pipeline(
            body,
            grid=(num_indices // _WINDOW,),
            in_specs=[pl.BlockSpec((1, _WINDOW), index_map=lambda i: (0, i))],
            out_specs=[pl.BlockSpec((_WINDOW, dim), index_map=lambda i: (i, 0))],
            core_axis_name=("core", "subcore"),
            dimension_semantics=(pltpu.PARALLEL,),
        )(i_hbm, o_hbm)

    return gather_kernel


def kernel(x, W):
    b, s = x.shape
    num_indices = b * s
    idx = x.reshape(1, num_indices).astype(jnp.int32)
    out = _build(num_indices, W.shape[1], W.dtype)(W, idx)
    return out.reshape(b, s, W.shape[1])


# speedup vs baseline: 1.4659x; 1.4659x over previous
import functools

import jax
import jax.numpy as jnp
from jax.experimental import pallas as pl
from jax.experimental.pallas import tpu as pltpu
from jax.experimental.pallas import tpu_sc as plsc

_WINDOW = 128


@functools.cache
def _build(num_indices: int, dim: int, dtype):
    mesh = plsc.VectorSubcoreMesh(core_axis_name="core",
                                  subcore_axis_name="subcore")

    @functools.partial(
        pl.kernel,
        out_type=jax.ShapeDtypeStruct((num_indices, dim), dtype),
        mesh=mesh,
        compiler_params=pltpu.CompilerParams(use_tc_tiling_on_sc=False),
    )
    def gather_kernel(w_hbm, i_hbm, o_hbm):
        def body(i_vmem, o_vmem):
            pltpu.sync_copy(w_hbm.at[i_vmem.at[0]], o_vmem)

        pltpu.emit_pipeline(
            body,
            grid=(num_indices // _WINDOW,),
            in_specs=[pl.BlockSpec((1, _WINDOW), index_map=lambda i: (0, i))],
            out_specs=[pl.BlockSpec((_WINDOW, dim), index_map=lambda i: (i, 0))],
            core_axis_name=("core", "subcore"),
            dimension_semantics=(pltpu.PARALLEL,),
        )(i_hbm, o_hbm)

    return gather_kernel


def kernel(x, W):
    b, s = x.shape
    num_indices = b * s
    idx = x.reshape(1, num_indices).astype(jnp.int32)
    out = _build(num_indices, W.shape[1], W.dtype)(W, idx)
    return out.reshape(b, s, W.shape[1])


# speedup vs baseline: 1.5517x; 1.0585x over previous
import functools

import jax
import jax.numpy as jnp
from jax.experimental import pallas as pl
from jax.experimental.pallas import tpu as pltpu
from jax.experimental.pallas import tpu_sc as plsc

_WINDOW = 512
_SUB = 128


@functools.cache
def _build(num_indices: int, dim: int, dtype):
    mesh = plsc.VectorSubcoreMesh(core_axis_name="core",
                                  subcore_axis_name="subcore")
    nsub = _WINDOW // _SUB

    @functools.partial(
        pl.kernel,
        out_type=jax.ShapeDtypeStruct((num_indices, dim), dtype),
        scratch_types=[pltpu.SemaphoreType.DMA],
        mesh=mesh,
        compiler_params=pltpu.CompilerParams(use_tc_tiling_on_sc=False),
    )
    def gather_kernel(w_hbm, i_hbm, o_hbm, sem):
        def body(i_vmem, o_vmem):
            for m in range(nsub):
                pltpu.async_copy(
                    w_hbm.at[i_vmem.at[0, pl.ds(m * _SUB, _SUB)]],
                    o_vmem.at[pl.ds(m * _SUB, _SUB)], sem)
            for m in range(nsub):
                pltpu.make_async_copy(
                    w_hbm.at[i_vmem.at[0, pl.ds(m * _SUB, _SUB)]],
                    o_vmem.at[pl.ds(m * _SUB, _SUB)], sem).wait()

        pltpu.emit_pipeline(
            body,
            grid=(num_indices // _WINDOW,),
            in_specs=[pl.BlockSpec((1, _WINDOW), index_map=lambda i: (0, i))],
            out_specs=[pl.BlockSpec((_WINDOW, dim), index_map=lambda i: (i, 0))],
            core_axis_name=("core", "subcore"),
            dimension_semantics=(pltpu.PARALLEL,),
        )(i_hbm, o_hbm)

    return gather_kernel


def kernel(x, W):
    b, s = x.shape
    num_indices = b * s
    idx = x.reshape(1, num_indices).astype(jnp.int32)
    out = _build(num_indices, W.shape[1], W.dtype)(W, idx)
    return out.reshape(b, s, W.shape[1])


# speedup vs baseline: 1.5627x; 1.0071x over previous
import functools

import jax
from jax import lax
import jax.numpy as jnp
from jax.experimental import pallas as pl
from jax.experimental.pallas import tpu as pltpu
from jax.experimental.pallas import tpu_sc as plsc

_CHUNK = 128
_NW = 32


@functools.cache
def _build(num_indices: int, out_rows: int, dim: int, dtype):
    mesh = plsc.VectorSubcoreMesh(core_axis_name="core",
                                  subcore_axis_name="subcore")
    per_w = num_indices // _NW
    nchunks = per_w // _CHUNK

    @functools.partial(
        pl.kernel,
        out_type=jax.ShapeDtypeStruct((out_rows, dim), dtype),
        scratch_types=[
            pltpu.VMEM((1, _CHUNK), jnp.int32),
            pltpu.VMEM((1, _CHUNK), jnp.int32),
            pltpu.VMEM((_CHUNK, dim), dtype),
            pltpu.SemaphoreType.DMA,
            pltpu.SemaphoreType.DMA,
        ],
        mesh=mesh,
        compiler_params=pltpu.CompilerParams(use_tc_tiling_on_sc=False),
    )
    def gather_kernel(w_hbm, i_hbm, r_hbm, o_hbm, idx_v, ridx_v, rows_v,
                      sem_g, sem_s):
        wid = lax.axis_index("subcore") * 2 + lax.axis_index("core")
        base = wid * per_w

        @pl.loop(0, nchunks)
        def _(c):
            off = base + c * _CHUNK
            pltpu.sync_copy(i_hbm.at[0, pl.ds(off, _CHUNK)], idx_v.at[0])
            pltpu.sync_copy(r_hbm.at[0, pl.ds(off, _CHUNK)], ridx_v.at[0])
            pltpu.async_copy(w_hbm.at[idx_v.at[0]], rows_v, sem_g).wait()
            pltpu.async_copy(rows_v, o_hbm.at[ridx_v.at[0]], sem_s).wait()

    return gather_kernel


def kernel(x, W):
    b, s = x.shape
    num_indices = b * s
    dim = W.shape[1]
    lanes = 128 // dim
    s_pad = (s + 7) // 8 * 8
    out_rows = b * s_pad * lanes

    idx = x.reshape(1, num_indices).astype(jnp.int32)
    t = jnp.arange(num_indices, dtype=jnp.int32)
    ridx = (((t // s) * s_pad + t % s) * lanes).reshape(1, num_indices)

    out = _build(num_indices, out_rows, dim, W.dtype)(W, idx, ridx)
    out = out.reshape(b, s_pad, lanes * dim)[:, :s, :dim]
    return out


# speedup vs baseline: 1.9769x; 1.2650x over previous
import functools

import jax
from jax import lax
import jax.numpy as jnp
from jax.experimental import pallas as pl
from jax.experimental.pallas import tpu as pltpu
from jax.experimental.pallas import tpu_sc as plsc

_CHUNK = 128
_NBUF = 8
_AHEAD = 4
_NW = 32


@functools.cache
def _build(num_indices: int, out_rows: int, dim: int, dtype):
    mesh = plsc.VectorSubcoreMesh(core_axis_name="core",
                                  subcore_axis_name="subcore")
    per_w = num_indices // _NW
    nchunks = per_w // _CHUNK

    @functools.partial(
        pl.kernel,
        out_type=jax.ShapeDtypeStruct((out_rows, dim), dtype),
        scratch_types=[
            pltpu.VMEM((1, per_w), jnp.int32),
            pltpu.VMEM((1, per_w), jnp.int32),
            pltpu.VMEM((_NBUF * _CHUNK, dim), dtype),
            pltpu.SemaphoreType.DMA,
            pltpu.SemaphoreType.DMA,
        ],
        mesh=mesh,
        compiler_params=pltpu.CompilerParams(use_tc_tiling_on_sc=False),
    )
    def gather_kernel(w_hbm, i_hbm, r_hbm, o_hbm, idx_v, ridx_v, rows_v,
                      sem_g, sem_s):
        wid = lax.axis_index("subcore") * 2 + lax.axis_index("core")
        base = wid * per_w

        pltpu.sync_copy(i_hbm.at[0, pl.ds(base, per_w)], idx_v.at[0])
        pltpu.sync_copy(r_hbm.at[0, pl.ds(base, per_w)], ridx_v.at[0])

        def gather(c):
            slot = lax.rem(c, _NBUF)
            return pltpu.make_async_copy(
                w_hbm.at[idx_v.at[0, pl.ds(c * _CHUNK, _CHUNK)]],
                rows_v.at[pl.ds(slot * _CHUNK, _CHUNK)], sem_g)

        def scatter(c):
            slot = lax.rem(c, _NBUF)
            return pltpu.make_async_copy(
                rows_v.at[pl.ds(slot * _CHUNK, _CHUNK)],
                o_hbm.at[ridx_v.at[0, pl.ds(c * _CHUNK, _CHUNK)]], sem_s)

        for c in range(_AHEAD):
            gather(c).start()

        @pl.loop(0, nchunks)
        def _(c):
            gather(c).wait()
            scatter(c).start()

            @pl.when(c >= _AHEAD)
            def _():
                scatter(c - _AHEAD).wait()

            @pl.when(c < nchunks - _AHEAD)
            def _():
                gather(c + _AHEAD).start()

        @pl.loop(nchunks - _AHEAD, nchunks)
        def _(c):
            scatter(c).wait()

    return gather_kernel


def kernel(x, W):
    b, s = x.shape
    num_indices = b * s
    dim = W.shape[1]
    lanes = 128 // dim
    s_pad = (s + 7) // 8 * 8
    out_rows = b * s_pad * lanes

    idx = x.reshape(1, num_indices).astype(jnp.int32)
    t = jnp.arange(num_indices, dtype=jnp.int32)
    ridx = (((t // s) * s_pad + t % s) * lanes).reshape(1, num_indices)

    out = _build(num_indices, out_rows, dim, W.dtype)(W, idx, ridx)
    out = out.reshape(b, s_pad, lanes * dim)[:, :s, :dim]
    return out


# speedup vs baseline: 2.5681x; 1.2990x over previous
import functools

import jax
from jax import lax
import jax.numpy as jnp
from jax.experimental import pallas as pl
from jax.experimental.pallas import tpu as pltpu
from jax.experimental.pallas import tpu_sc as plsc

_CHUNK = 128
_NBUF = 8
_AHEAD = 4
_NW = 32


@functools.cache
def _build(num_indices: int, out_rows: int, dim: int, dtype):
    mesh = plsc.VectorSubcoreMesh(core_axis_name="core",
                                  subcore_axis_name="subcore")
    per_w = num_indices // _NW
    nchunks = per_w // _CHUNK

    @functools.partial(
        pl.kernel,
        out_type=jax.ShapeDtypeStruct((out_rows, dim), dtype),
        scratch_types=[
            pltpu.VMEM((1, per_w), jnp.int32),
            pltpu.VMEM((1, per_w), jnp.int32),
            pltpu.VMEM((_NBUF * _CHUNK, dim), dtype),
            pltpu.SemaphoreType.DMA,
            pltpu.SemaphoreType.DMA,
        ],
        mesh=mesh,
        compiler_params=pltpu.CompilerParams(use_tc_tiling_on_sc=False),
    )
    def gather_kernel(w_hbm, i_hbm, r_hbm, o_hbm, idx_v, ridx_v, rows_v,
                      sem_g, sem_s):
        wid = lax.axis_index("subcore") * 2 + lax.axis_index("core")
        base = wid * per_w

        pltpu.sync_copy(i_hbm.at[0, pl.ds(base, per_w)], idx_v.at[0])
        pltpu.sync_copy(r_hbm.at[0, pl.ds(base, per_w)], ridx_v.at[0])

        def gather(c):
            slot = lax.rem(c, _NBUF)
            return pltpu.make_async_copy(
                w_hbm.at[idx_v.at[0, pl.ds(c * _CHUNK, _CHUNK)]],
                rows_v.at[pl.ds(slot * _CHUNK, _CHUNK)], sem_g)

        def scatter(c):
            slot = lax.rem(c, _NBUF)
            return pltpu.make_async_copy(
                rows_v.at[pl.ds(slot * _CHUNK, _CHUNK)],
                o_hbm.at[ridx_v.at[0, pl.ds(c * _CHUNK, _CHUNK)]], sem_s)

        for c in range(_AHEAD):
            gather(c).start()

        @pl.loop(0, nchunks)
        def _(c):
            gather(c).wait()
            scatter(c).start()

            @pl.when(c >= _AHEAD)
            def _():
                scatter(c - _AHEAD).wait()

            @pl.when(c < nchunks - _AHEAD)
            def _():
                gather(c + _AHEAD).start()

        @pl.loop(nchunks - _AHEAD, nchunks)
        def _(c):
            scatter(c).wait()

    return gather_kernel


_TBLK = 4096


@functools.cache
def _build_transpose(vocab: int, dim: int, dtype):
    lanes = 128 // dim
    blk_out = _TBLK // lanes
    grid = (vocab + _TBLK - 1) // _TBLK
    rows_out = grid * blk_out

    def tbody(i_ref, o_ref):
        x = i_ref[...]
        for q in range(lanes):
            o_ref[:, 32 * q:32 * (q + 1)] = x[:, blk_out * q:blk_out * (q + 1)].T

    return pl.pallas_call(
        tbody,
        grid=(grid,),
        in_specs=[pl.BlockSpec((dim, _TBLK), lambda c: (0, c))],
        out_specs=pl.BlockSpec((blk_out, 128), lambda c: (c, 0)),
        out_shape=jax.ShapeDtypeStruct((rows_out, 128), dtype),
        compiler_params=pltpu.CompilerParams(
            dimension_semantics=("arbitrary",)),
    )


def kernel(x, W):
    b, s = x.shape
    num_indices = b * s
    dim = W.shape[1]
    lanes = 128 // dim
    s_pad = (s + 7) // 8 * 8
    out_rows = b * s_pad * lanes

    v = x.reshape(1, num_indices).astype(jnp.int32)
    blk_out = _TBLK // lanes
    idx = (lanes * (blk_out * (v // _TBLK) + v % blk_out)
           + (v % _TBLK) // blk_out)
    t = jnp.arange(num_indices, dtype=jnp.int32)
    ridx = (((t // s) * s_pad + t % s) * lanes).reshape(1, num_indices)

    vocab = W.shape[0]
    w128 = _build_transpose(vocab, dim, W.dtype)(W.T)
    w_lin = w128.reshape(w128.shape[0] * lanes, dim)
    out = _build(num_indices, out_rows, dim, W.dtype)(w_lin, idx, ridx)
    out = out.reshape(b, s_pad, lanes * dim)[:, :s, :dim]
    return out


# speedup vs baseline: 2.8446x; 1.1077x over previous
import functools

import jax
from jax import lax
import jax.numpy as jnp
from jax.experimental import pallas as pl
from jax.experimental.pallas import tpu as pltpu
from jax.experimental.pallas import tpu_sc as plsc

_CHUNK = 128
_NBUF = 8
_AHEAD = 4
_NW = 32


@functools.cache
def _build(num_indices: int, out_rows: int, dim: int, dtype):
    mesh = plsc.VectorSubcoreMesh(core_axis_name="core",
                                  subcore_axis_name="subcore")
    per_w = num_indices // _NW
    nchunks = per_w // _CHUNK

    @functools.partial(
        pl.kernel,
        out_type=jax.ShapeDtypeStruct((out_rows, dim), dtype),
        scratch_types=[
            pltpu.VMEM((1, per_w), jnp.int32),
            pltpu.VMEM((1, per_w), jnp.int32),
            pltpu.VMEM((_NBUF * _CHUNK, dim), dtype),
            pltpu.SemaphoreType.DMA,
            pltpu.SemaphoreType.DMA,
        ],
        mesh=mesh,
        compiler_params=pltpu.CompilerParams(use_tc_tiling_on_sc=False),
    )
    def gather_kernel(w_hbm, i_hbm, r_hbm, o_hbm, idx_v, ridx_v, rows_v,
                      sem_g, sem_s):
        wid = lax.axis_index("subcore") * 2 + lax.axis_index("core")
        base = wid * per_w

        pltpu.sync_copy(i_hbm.at[0, pl.ds(base, per_w)], idx_v.at[0])
        pltpu.sync_copy(r_hbm.at[0, pl.ds(base, per_w)], ridx_v.at[0])

        def gather(c):
            slot = lax.rem(c, _NBUF)
            return pltpu.make_async_copy(
                w_hbm.at[idx_v.at[0, pl.ds(c * _CHUNK, _CHUNK)]],
                rows_v.at[pl.ds(slot * _CHUNK, _CHUNK)], sem_g)

        def scatter(c):
            slot = lax.rem(c, _NBUF)
            return pltpu.make_async_copy(
                rows_v.at[pl.ds(slot * _CHUNK, _CHUNK)],
                o_hbm.at[ridx_v.at[0, pl.ds(c * _CHUNK, _CHUNK)]], sem_s)

        for c in range(_AHEAD):
            gather(c).start()

        @pl.loop(0, nchunks)
        def _(c):
            gather(c).wait()
            scatter(c).start()

            @pl.when(c >= _AHEAD)
            def _():
                scatter(c - _AHEAD).wait()

            @pl.when(c < nchunks - _AHEAD)
            def _():
                gather(c + _AHEAD).start()

        @pl.loop(nchunks - _AHEAD, nchunks)
        def _(c):
            scatter(c).wait()

    return gather_kernel


_TBLK = 16384


@functools.cache
def _build_transpose(vocab: int, dim: int, dtype):
    lanes = 128 // dim
    blk_out = _TBLK // lanes
    grid = (vocab + _TBLK - 1) // _TBLK
    rows_out = grid * blk_out

    def tbody(i_ref, o_ref):
        x = i_ref[...]
        for q in range(lanes):
            o_ref[:, 32 * q:32 * (q + 1)] = x[:, blk_out * q:blk_out * (q + 1)].T

    return pl.pallas_call(
        tbody,
        grid=(grid,),
        in_specs=[pl.BlockSpec((dim, _TBLK), lambda c: (0, c))],
        out_specs=pl.BlockSpec((blk_out, 128), lambda c: (c, 0)),
        out_shape=jax.ShapeDtypeStruct((rows_out, 128), dtype),
        compiler_params=pltpu.CompilerParams(
            dimension_semantics=("arbitrary",)),
    )


def kernel(x, W):
    b, s = x.shape
    num_indices = b * s
    dim = W.shape[1]
    lanes = 128 // dim
    s_pad = (s + 7) // 8 * 8
    out_rows = b * s_pad * lanes

    v = x.reshape(1, num_indices).astype(jnp.int32)
    blk_out = _TBLK // lanes
    idx = (lanes * (blk_out * (v // _TBLK) + v % blk_out)
           + (v % _TBLK) // blk_out)
    t = jnp.arange(num_indices, dtype=jnp.int32)
    ridx = (((t // s) * s_pad + t % s) * lanes).reshape(1, num_indices)

    vocab = W.shape[0]
    w128 = _build_transpose(vocab, dim, W.dtype)(W.T)
    w_lin = w128.reshape(w128.shape[0] * lanes, dim)
    out = _build(num_indices, out_rows, dim, W.dtype)(w_lin, idx, ridx)
    out = out.reshape(b, s_pad, lanes * dim)[:, :s, :dim]
    return out


# speedup vs baseline: 2.8584x; 1.0049x over previous
import functools

import jax
from jax import lax
import jax.numpy as jnp
from jax.experimental import pallas as pl
from jax.experimental.pallas import tpu as pltpu
from jax.experimental.pallas import tpu_sc as plsc

_CHUNK = 128
_NBUF = 8
_AHEAD = 4
_NW = 32


@functools.cache
def _build(num_indices: int, out_rows: int, dim: int, dtype):
    mesh = plsc.VectorSubcoreMesh(core_axis_name="core",
                                  subcore_axis_name="subcore")
    per_w = num_indices // _NW
    nchunks = per_w // _CHUNK

    @functools.partial(
        pl.kernel,
        out_type=jax.ShapeDtypeStruct((out_rows, dim), dtype),
        scratch_types=[
            pltpu.VMEM((1, per_w), jnp.int32),
            pltpu.VMEM((1, per_w), jnp.int32),
            pltpu.VMEM((_NBUF * _CHUNK, dim), dtype),
            pltpu.SemaphoreType.DMA,
            pltpu.SemaphoreType.DMA,
        ],
        mesh=mesh,
        compiler_params=pltpu.CompilerParams(use_tc_tiling_on_sc=False),
    )
    def gather_kernel(w_hbm, i_hbm, r_hbm, o_hbm, idx_v, ridx_v, rows_v,
                      sem_g, sem_s):
        wid = lax.axis_index("subcore") * 2 + lax.axis_index("core")
        base = wid * per_w

        pltpu.sync_copy(i_hbm.at[0, pl.ds(base, per_w)], idx_v.at[0])
        pltpu.sync_copy(r_hbm.at[0, pl.ds(base, per_w)], ridx_v.at[0])

        def gather(c):
            slot = lax.rem(c, _NBUF)
            return pltpu.make_async_copy(
                w_hbm.at[idx_v.at[0, pl.ds(c * _CHUNK, _CHUNK)]],
                rows_v.at[pl.ds(slot * _CHUNK, _CHUNK)], sem_g)

        def scatter(c):
            slot = lax.rem(c, _NBUF)
            return pltpu.make_async_copy(
                rows_v.at[pl.ds(slot * _CHUNK, _CHUNK)],
                o_hbm.at[ridx_v.at[0, pl.ds(c * _CHUNK, _CHUNK)]], sem_s)

        for c in range(_AHEAD):
            gather(c).start()

        @pl.loop(0, nchunks)
        def _(c):
            gather(c).wait()
            scatter(c).start()

            @pl.when(c >= _AHEAD)
            def _():
                scatter(c - _AHEAD).wait()

            @pl.when(c < nchunks - _AHEAD)
            def _():
                gather(c + _AHEAD).start()

        @pl.loop(nchunks - _AHEAD, nchunks)
        def _(c):
            scatter(c).wait()

    return gather_kernel


_TBLK = 32768


@functools.cache
def _build_transpose(vocab: int, dim: int, dtype):
    lanes = 128 // dim
    blk_out = _TBLK // lanes
    grid = (vocab + _TBLK - 1) // _TBLK
    rows_out = grid * blk_out

    def tbody(i_ref, o_ref):
        x = i_ref[...]
        for q in range(lanes):
            o_ref[:, 32 * q:32 * (q + 1)] = x[:, blk_out * q:blk_out * (q + 1)].T

    return pl.pallas_call(
        tbody,
        grid=(grid,),
        in_specs=[pl.BlockSpec((dim, _TBLK), lambda c: (0, c))],
        out_specs=pl.BlockSpec((blk_out, 128), lambda c: (c, 0)),
        out_shape=jax.ShapeDtypeStruct((rows_out, 128), dtype),
        compiler_params=pltpu.CompilerParams(
            dimension_semantics=("arbitrary",)),
    )


def kernel(x, W):
    b, s = x.shape
    num_indices = b * s
    dim = W.shape[1]
    lanes = 128 // dim
    s_pad = (s + 7) // 8 * 8
    out_rows = b * s_pad * lanes

    v = x.reshape(1, num_indices).astype(jnp.int32)
    blk_out = _TBLK // lanes
    idx = (lanes * (blk_out * (v // _TBLK) + v % blk_out)
           + (v % _TBLK) // blk_out)
    t = jnp.arange(num_indices, dtype=jnp.int32)
    ridx = (((t // s) * s_pad + t % s) * lanes).reshape(1, num_indices)

    vocab = W.shape[0]
    w128 = _build_transpose(vocab, dim, W.dtype)(W.T)
    w_lin = w128.reshape(w128.shape[0] * lanes, dim)
    out = _build(num_indices, out_rows, dim, W.dtype)(w_lin, idx, ridx)
    out = out.reshape(b, s_pad, lanes * dim)[:, :s, :dim]
    return out
